# Initial kernel scaffold; baseline (speedup 1.0000x reference)
import jax
import jax.numpy as jnp
from jax.experimental import pallas as pl


def kernel(verts, deltaG, origin, spacing):
    raise NotImplementedError("write your pallas kernel here")


# speedup vs baseline: 365.4350x; 365.4350x over previous
import functools

import jax
import jax.numpy as jnp
from jax import lax
from jax.experimental import pallas as pl
from jax.experimental.pallas import tpu as pltpu
from jax.experimental.pallas import tpu_sc as plsc

_NX = _NY = _NZ = 64
_PX, _PY, _PZ = _NX + 1, _NY + 1, _NZ + 1
_B = 128
_L = 16
_NW = 32
_GRP = _B // _L


def _sc_ffd(n_pad, blocks_per_worker):
    mesh = plsc.VectorSubcoreMesh(core_axis_name="c", subcore_axis_name="s")
    fdt = jnp.float32
    idt = jnp.int32
    out_t = [jax.ShapeDtypeStruct((n_pad,), fdt)] * 3
    scratch = (
        [pltpu.VMEM((96,), fdt)]
        + [pltpu.VMEM((_B,), fdt)] * 3
        + [pltpu.VMEM((_B,), idt) for _ in range(16)]
        + [pltpu.VMEM((_B, _L), fdt) for _ in range(16)]
        + [pltpu.VMEM((_B,), fdt)] * 3
        + [pltpu.SemaphoreType.DMA]
    )

    @functools.partial(
        pl.kernel, mesh=mesh, out_type=out_t, scratch_types=scratch,
        compiler_params=pltpu.CompilerParams(
            needs_layout_passes=False, use_tc_tiling_on_sc=False),
    )
    def k(e_hbm, vx_hbm, vy_hbm, vz_hbm, par_hbm, ox_hbm, oy_hbm, oz_hbm,
          *refs):
        pbuf = refs[0]
        vb = refs[1:4]
        idxb = refs[4:20]
        rowb = refs[20:36]
        ob = refs[36:39]
        sem = refs[39]

        wid = lax.axis_index("s") * 2 + lax.axis_index("c")
        chunk = blocks_per_worker * _B
        pltpu.sync_copy(par_hbm, pbuf)
        lanes = lax.iota(idt, _L)

        def cell(p, o, s):
            rel = (p - o) * s
            b = rel.astype(idt)
            b = jnp.minimum(jnp.maximum(b, 0), 61)
            return b, rel - b.astype(fdt)

        def basis(u):
            u2 = u * u
            u3 = u2 * u
            return (
                (1.0 - 3.0 * u + 3.0 * u2 - u3) * (1.0 / 6.0),
                (4.0 - 6.0 * u2 + 3.0 * u3) * (1.0 / 6.0),
                (1.0 + 3.0 * u + 3.0 * u2 - 3.0 * u3) * (1.0 / 6.0),
                u3 * (1.0 / 6.0),
            )

        def block(blk, _):
            base = wid * chunk + blk * _B
            for src, dst in zip((vx_hbm, vy_hbm, vz_hbm), vb):
                pltpu.sync_copy(src.at[pl.ds(base, _B)], dst)

            ox = pbuf[pl.ds(0, _L)]
            oy = pbuf[pl.ds(16, _L)]
            oz = pbuf[pl.ds(32, _L)]
            sx = pbuf[pl.ds(48, _L)]
            sy = pbuf[pl.ds(64, _L)]
            sz = pbuf[pl.ds(80, _L)]

            def idx_grp(g, _):
                s = pl.ds(g * _L, _L)
                bx, _u = cell(vb[0][s], ox, sx)
                by, _u = cell(vb[1][s], oy, sy)
                bz, _u = cell(vb[2][s], oz, sz)
                t = (bx * _PY + by) * _PZ + bz
                for a in range(4):
                    for b in range(4):
                        idxb[a * 4 + b][s] = t + (a * _PY + b) * _PZ
                return _
            lax.fori_loop(0, _GRP, idx_grp, None)

            handles = [
                pltpu.async_copy(e_hbm.at[idxb[i]], rowb[i], sem)
                for i in range(16)
            ]
            for h in handles:
                h.wait()

            def fma_grp(g, _):
                s = pl.ds(g * _L, _L)
                px, py, pz = vb[0][s], vb[1][s], vb[2][s]
                _b, ux = cell(px, ox, sx)
                _b, uy = cell(py, oy, sy)
                _b, uz = cell(pz, oz, sz)
                bu = basis(ux)
                bv = basis(uy)
                bw = basis(uz)
                jv = lanes + g * _L
                acc = [px, py, pz]
                for a in range(4):
                    for b in range(4):
                        wab = bu[a] * bv[b]
                        r = rowb[a * 4 + b]
                        for c in range(4):
                            wabc = wab * bw[c]
                            for d in range(3):
                                kv = jnp.full((_L,), 3 * c + d, idt)
                                g16 = plsc.load_gather(r, [jv, kv])
                                acc[d] = acc[d] + wabc * g16
                for d in range(3):
                    ob[d][s] = acc[d]
                return _
            lax.fori_loop(0, _GRP, fma_grp, None)

            for dst, src in zip((ox_hbm, oy_hbm, oz_hbm), ob):
                pltpu.sync_copy(src, dst.at[pl.ds(base, _B)])
            return _

        lax.fori_loop(0, blocks_per_worker, block, None)

    return k


def kernel(verts, deltaG, origin, spacing):
    n = verts.shape[0]
    step = _NW * _B
    n_pad = ((n + step - 1) // step) * step
    bpw = n_pad // step

    pad = jnp.pad(deltaG.astype(jnp.float32),
                  ((1, 0), (1, 0), (1, 3), (0, 0)))
    e12 = jnp.concatenate([pad[:, :, c:c + _PZ, :] for c in range(4)],
                          axis=-1)
    e = jnp.pad(e12, ((0, 0), (0, 0), (0, 0), (0, 4)))
    e = e.reshape(_PX * _PY * _PZ, 16)

    vx = jnp.pad(verts[:, 0], (0, n_pad - n))
    vy = jnp.pad(verts[:, 1], (0, n_pad - n))
    vz = jnp.pad(verts[:, 2], (0, n_pad - n))

    inv_sp = 1.0 / spacing.astype(jnp.float32)
    par = jnp.concatenate([
        jnp.broadcast_to(origin.astype(jnp.float32)[i], (16,))
        for i in range(3)
    ] + [jnp.broadcast_to(inv_sp[i], (16,)) for i in range(3)])

    ox, oy, oz = _sc_ffd(n_pad, bpw)(e, vx, vy, vz, par)
    return jnp.stack([ox[:n], oy[:n], oz[:n]], axis=1)


# speedup vs baseline: 550.9506x; 1.5077x over previous
import functools

import jax
import jax.numpy as jnp
from jax import lax
from jax.experimental import pallas as pl
from jax.experimental.pallas import tpu as pltpu
from jax.experimental.pallas import tpu_sc as plsc

_NX = _NY = _NZ = 64
_PX, _PY, _PZ = _NX + 1, _NY + 1, _NZ + 1
_B = 128
_L = 16
_NW = 32
_GRP = _B // _L


def _sc_ffd(n_pad, blocks_per_worker):
    mesh = plsc.VectorSubcoreMesh(core_axis_name="c", subcore_axis_name="s")
    fdt = jnp.float32
    idt = jnp.int32
    out_t = [jax.ShapeDtypeStruct((n_pad,), fdt)] * 3
    one_set = (
        [pltpu.VMEM((_B,), fdt)] * 3
        + [pltpu.VMEM((_B,), idt) for _ in range(16)]
        + [pltpu.VMEM((_B, _L), fdt) for _ in range(16)]
        + [pltpu.SemaphoreType.DMA]
    )
    scratch = (
        [pltpu.VMEM((96,), fdt)]
        + [pltpu.VMEM((_B,), fdt)] * 3
        + one_set + one_set
    )

    @functools.partial(
        pl.kernel, mesh=mesh, out_type=out_t, scratch_types=scratch,
        compiler_params=pltpu.CompilerParams(
            needs_layout_passes=False, use_tc_tiling_on_sc=False),
    )
    def k(e_hbm, vx_hbm, vy_hbm, vz_hbm, par_hbm, ox_hbm, oy_hbm, oz_hbm,
          *refs):
        pbuf = refs[0]
        ob = refs[1:4]
        sets = []
        for i in range(2):
            s0 = 4 + i * 36
            sets.append(dict(
                vb=refs[s0:s0 + 3],
                idxb=refs[s0 + 3:s0 + 19],
                rowb=refs[s0 + 19:s0 + 35],
                sem=refs[s0 + 35],
            ))

        wid = lax.axis_index("s") * 2 + lax.axis_index("c")
        chunk = blocks_per_worker * _B
        pltpu.sync_copy(par_hbm, pbuf)
        lanes = lax.iota(idt, _L)

        def par(i):
            return pbuf[pl.ds(16 * i, _L)]

        def cell(p, o, s):
            rel = (p - o) * s
            b = rel.astype(idt)
            b = jnp.minimum(jnp.maximum(b, 0), 61)
            return b, rel - b.astype(fdt)

        def basis(u):
            u2 = u * u
            u3 = u2 * u
            return (
                (1.0 - 3.0 * u + 3.0 * u2 - u3) * (1.0 / 6.0),
                (4.0 - 6.0 * u2 + 3.0 * u3) * (1.0 / 6.0),
                (1.0 + 3.0 * u + 3.0 * u2 - 3.0 * u3) * (1.0 / 6.0),
                u3 * (1.0 / 6.0),
            )

        def stage(blk, s):
            base = wid * chunk + blk * _B
            for src, dst in zip((vx_hbm, vy_hbm, vz_hbm), s["vb"]):
                pltpu.sync_copy(src.at[pl.ds(base, _B)], dst)

            def idx_grp(g, _):
                sl = pl.ds(g * _L, _L)
                bx, _u = cell(s["vb"][0][sl], par(0), par(3))
                by, _u = cell(s["vb"][1][sl], par(1), par(4))
                bz, _u = cell(s["vb"][2][sl], par(2), par(5))
                t = (bx * _PY + by) * _PZ + bz
                for a in range(4):
                    for b in range(4):
                        s["idxb"][a * 4 + b][sl] = t + (a * _PY + b) * _PZ
                return _
            lax.fori_loop(0, _GRP, idx_grp, None)
            for i in range(16):
                pltpu.async_copy(e_hbm.at[s["idxb"][i]], s["rowb"][i],
                                 s["sem"])

        def drain(s):
            for i in range(16):
                pltpu.make_async_copy(e_hbm.at[s["idxb"][i]], s["rowb"][i],
                                      s["sem"]).wait()

        def compute(blk, s):
            base = wid * chunk + blk * _B
            drain(s)

            def fma_grp(g, _):
                sl = pl.ds(g * _L, _L)
                px = s["vb"][0][sl]
                py = s["vb"][1][sl]
                pz = s["vb"][2][sl]
                _b, ux = cell(px, par(0), par(3))
                _b, uy = cell(py, par(1), par(4))
                _b, uz = cell(pz, par(2), par(5))
                bu = basis(ux)
                bv = basis(uy)
                bw = basis(uz)
                jv = lanes + g * _L
                acc = [px, py, pz]
                for a in range(4):
                    for b in range(4):
                        wab = bu[a] * bv[b]
                        r = s["rowb"][a * 4 + b]
                        for c in range(4):
                            wabc = wab * bw[c]
                            for d in range(3):
                                kv = jnp.full((_L,), 3 * c + d, idt)
                                g16 = plsc.load_gather(r, [jv, kv])
                                acc[d] = acc[d] + wabc * g16
                for d in range(3):
                    ob[d][sl] = acc[d]
                return _
            lax.fori_loop(0, _GRP, fma_grp, None)

            for dst, src in zip((ox_hbm, oy_hbm, oz_hbm), ob):
                pltpu.sync_copy(src, dst.at[pl.ds(base, _B)])

        stage(0, sets[0])

        def body(i, _):
            b0 = 2 * i
            stage(b0 + 1, sets[1])
            compute(b0, sets[0])

            @pl.when(b0 + 2 < blocks_per_worker)
            def _fire_next():
                stage(b0 + 2, sets[0])

            compute(b0 + 1, sets[1])
            return _

        lax.fori_loop(0, blocks_per_worker // 2, body, None)

    return k


def kernel(verts, deltaG, origin, spacing):
    n = verts.shape[0]
    step = 2 * _NW * _B
    n_pad = ((n + step - 1) // step) * step
    bpw = n_pad // (_NW * _B)

    pad = jnp.pad(deltaG.astype(jnp.float32),
                  ((1, 0), (1, 0), (1, 3), (0, 0)))
    e12 = jnp.concatenate([pad[:, :, c:c + _PZ, :] for c in range(4)],
                          axis=-1)
    e = jnp.pad(e12, ((0, 0), (0, 0), (0, 0), (0, 4)))
    e = e.reshape(_PX * _PY * _PZ, 16)

    vx = jnp.pad(verts[:, 0], (0, n_pad - n))
    vy = jnp.pad(verts[:, 1], (0, n_pad - n))
    vz = jnp.pad(verts[:, 2], (0, n_pad - n))

    inv_sp = 1.0 / spacing.astype(jnp.float32)
    par = jnp.concatenate([
        jnp.broadcast_to(origin.astype(jnp.float32)[i], (16,))
        for i in range(3)
    ] + [jnp.broadcast_to(inv_sp[i], (16,)) for i in range(3)])

    ox, oy, oz = _sc_ffd(n_pad, bpw)(e, vx, vy, vz, par)
    return jnp.stack([ox[:n], oy[:n], oz[:n]], axis=1)


# speedup vs baseline: 654.8920x; 1.1887x over previous
import functools

import jax
import jax.numpy as jnp
from jax import lax
from jax.experimental import pallas as pl
from jax.experimental.pallas import tpu as pltpu
from jax.experimental.pallas import tpu_sc as plsc

_NX = _NY = _NZ = 64
_PX, _PY, _PZ = _NX + 1, _NY + 1, _NZ + 1
_B = 128
_L = 16
_NW = 32
_GRP = _B // _L


def _sc_ffd(n_pad, blocks_per_worker):
    mesh = plsc.VectorSubcoreMesh(core_axis_name="c", subcore_axis_name="s")
    fdt = jnp.float32
    idt = jnp.int32
    nblk = n_pad // _B
    out_t = jax.ShapeDtypeStruct((nblk, 3 * _B), fdt)
    one_set = (
        [pltpu.VMEM((3 * _B,), fdt)]
        + [pltpu.VMEM((_B,), idt) for _ in range(16)]
        + [pltpu.VMEM((_B, _L), fdt) for _ in range(16)]
        + [pltpu.VMEM((3 * _B,), fdt)]
        + [pltpu.SemaphoreType.DMA, pltpu.SemaphoreType.DMA]
    )
    scratch = [pltpu.VMEM((96,), fdt)] + one_set + one_set

    @functools.partial(
        pl.kernel, mesh=mesh, out_type=out_t, scratch_types=scratch,
        compiler_params=pltpu.CompilerParams(
            needs_layout_passes=False, use_tc_tiling_on_sc=False),
    )
    def k(e_hbm, v_hbm, par_hbm, o_hbm, *refs):
        pbuf = refs[0]
        sets = []
        for i in range(2):
            s0 = 1 + i * 36
            sets.append(dict(
                vb=refs[s0],
                idxb=refs[s0 + 1:s0 + 17],
                rowb=refs[s0 + 17:s0 + 33],
                ob=refs[s0 + 33],
                sem=refs[s0 + 34],
                osem=refs[s0 + 35],
            ))

        wid = lax.axis_index("s") * 2 + lax.axis_index("c")
        pltpu.sync_copy(par_hbm, pbuf)
        lanes = lax.iota(idt, _L)

        def par(i):
            return pbuf[pl.ds(16 * i, _L)]

        def cell(p, o, s):
            rel = (p - o) * s
            b = rel.astype(idt)
            b = jnp.minimum(jnp.maximum(b, 0), 61)
            return b, rel - b.astype(fdt)

        def basis(u):
            u2 = u * u
            u3 = u2 * u
            return (
                (1.0 - 3.0 * u + 3.0 * u2 - u3) * (1.0 / 6.0),
                (4.0 - 6.0 * u2 + 3.0 * u3) * (1.0 / 6.0),
                (1.0 + 3.0 * u + 3.0 * u2 - 3.0 * u3) * (1.0 / 6.0),
                u3 * (1.0 / 6.0),
            )

        def stage(blk, s):
            gb = wid * blocks_per_worker + blk
            pltpu.sync_copy(v_hbm.at[gb], s["vb"])

            def idx_grp(g, _):
                sl = pl.ds(g * _L, _L)
                bx, _u = cell(s["vb"][pl.ds(g * _L, _L)], par(0), par(3))
                by, _u = cell(s["vb"][pl.ds(_B + g * _L, _L)], par(1), par(4))
                bz, _u = cell(s["vb"][pl.ds(2 * _B + g * _L, _L)],
                              par(2), par(5))
                t = (bx * _PY + by) * _PZ + bz
                for a in range(4):
                    for b in range(4):
                        s["idxb"][a * 4 + b][sl] = t + (a * _PY + b) * _PZ
                return _
            lax.fori_loop(0, _GRP, idx_grp, None)
            for i in range(16):
                pltpu.async_copy(e_hbm.at[s["idxb"][i]], s["rowb"][i],
                                 s["sem"])

        def compute(blk, s, has_out):
            gb = wid * blocks_per_worker + blk
            for i in range(16):
                pltpu.make_async_copy(e_hbm.at[s["idxb"][i]], s["rowb"][i],
                                      s["sem"]).wait()
            if has_out:
                pltpu.make_async_copy(s["ob"], o_hbm.at[gb],
                                      s["osem"]).wait()

            def fma_grp(g, _):
                px = s["vb"][pl.ds(g * _L, _L)]
                py = s["vb"][pl.ds(_B + g * _L, _L)]
                pz = s["vb"][pl.ds(2 * _B + g * _L, _L)]
                _b, ux = cell(px, par(0), par(3))
                _b, uy = cell(py, par(1), par(4))
                _b, uz = cell(pz, par(2), par(5))
                bu = basis(ux)
                bv = basis(uy)
                bw = basis(uz)
                jv = lanes + g * _L
                acc = [px, py, pz]
                for a in range(4):
                    for b in range(4):
                        wab = bu[a] * bv[b]
                        r = s["rowb"][a * 4 + b]
                        for c in range(4):
                            wabc = wab * bw[c]
                            for d in range(3):
                                kv = jnp.full((_L,), 3 * c + d, idt)
                                g16 = plsc.load_gather(r, [jv, kv])
                                acc[d] = acc[d] + wabc * g16
                for d in range(3):
                    s["ob"][pl.ds(d * _B + g * _L, _L)] = acc[d]
                return _
            lax.fori_loop(0, _GRP, fma_grp, None)
            pltpu.async_copy(s["ob"], o_hbm.at[gb], s["osem"])

        stage(0, sets[0])
        stage(1, sets[1])
        compute(0, sets[0], has_out=False)
        stage(2, sets[0])
        compute(1, sets[1], has_out=False)

        def body(i, _):
            b0 = 2 * i
            stage(b0 + 1, sets[1])
            compute(b0, sets[0], has_out=True)

            @pl.when(b0 + 2 < blocks_per_worker)
            def _fire_next():
                stage(b0 + 2, sets[0])

            compute(b0 + 1, sets[1], has_out=True)
            return _

        lax.fori_loop(1, blocks_per_worker // 2, body, None)
        for s in sets:
            pltpu.make_async_copy(s["ob"], o_hbm.at[0], s["osem"]).wait()

    return k


def kernel(verts, deltaG, origin, spacing):
    n = verts.shape[0]
    step = 2 * _NW * _B
    n_pad = max(((n + step - 1) // step) * step, 2 * step)
    bpw = n_pad // (_NW * _B)

    pad = jnp.pad(deltaG.astype(jnp.float32),
                  ((1, 0), (1, 0), (1, 3), (0, 0)))
    e12 = jnp.concatenate([pad[:, :, c:c + _PZ, :] for c in range(4)],
                          axis=-1)
    e = jnp.pad(e12, ((0, 0), (0, 0), (0, 0), (0, 4)))
    e = e.reshape(_PX * _PY * _PZ, 16)

    nblk = n_pad // _B
    vpack = jnp.pad(verts.astype(jnp.float32),
                    ((0, n_pad - n), (0, 0)))
    vpack = vpack.reshape(nblk, _B, 3).transpose(0, 2, 1).reshape(nblk, 3 * _B)

    inv_sp = 1.0 / spacing.astype(jnp.float32)
    par = jnp.concatenate([
        jnp.broadcast_to(origin.astype(jnp.float32)[i], (16,))
        for i in range(3)
    ] + [jnp.broadcast_to(inv_sp[i], (16,)) for i in range(3)])

    o = _sc_ffd(n_pad, bpw)(e, vpack, par)
    o = o.reshape(nblk, 3, _B).transpose(0, 2, 1).reshape(n_pad, 3)
    return o[:n]
